# Initial kernel scaffold; baseline (speedup 1.0000x reference)
import jax
import jax.numpy as jnp
from jax.experimental import pallas as pl


def kernel(x, adj_t, W1, b1, W2, b2, W3, b3):
    raise NotImplementedError("write your pallas kernel here")


# speedup vs baseline: 4.4485x; 4.4485x over previous
import functools

import jax
import jax.numpy as jnp
from jax import lax
from jax.experimental import pallas as pl
from jax.experimental.pallas import tpu as pltpu
from jax.experimental.pallas import tpu_sc as plsc

N = 10000
D = 256
H = 128
NC = 2
NS = 16
K = 128
EPT = 80
E_PAD = NS * EPT * K
ACC_ROWS = 10240
RB = 1000

_sc_mesh = plsc.VectorSubcoreMesh(
    core_axis_name="c", subcore_axis_name="s", num_cores=NC, num_subcores=NS)


_deg_kernel_args = dict(
    out_type=jax.ShapeDtypeStruct((NC * N, H), jnp.float32),
    mesh=_sc_mesh,
    scratch_types=[
        pltpu.VMEM((EPT, K), jnp.int32),
        pltpu.VMEM((K, H), jnp.float32),
        pltpu.VMEM((K, H), jnp.float32),
        pltpu.VMEM_SHARED((ACC_ROWS, H), jnp.float32),
    ],
)


def _sc_deg_body(dst_hbm, out_hbm, dstv, ones, zeros, acc):
    c = lax.axis_index("c")
    s = lax.axis_index("s")
    pltpu.sync_copy(dst_hbm.at[s], dstv)

    def fill(i, carry):
        ones[i // 8, pl.ds((i % 8) * 16, 16)] = jnp.ones((16,), jnp.float32)
        zeros[i // 8, pl.ds((i % 8) * 16, 16)] = jnp.zeros((16,), jnp.float32)
        return carry
    lax.fori_loop(0, K * H // 16, fill, None)
    for k in range(5):
        pltpu.sync_copy(zeros, acc.at[pl.ds(s * 640 + k * K, K)])
    plsc.subcore_barrier()

    def body(j, carry):
        pltpu.sync_copy(ones, acc.at[dstv.at[j]], add=True)
        return carry
    lax.fori_loop(0, EPT, body, None)
    plsc.subcore_barrier()

    @pl.when(s < N // RB)
    def _():
        pltpu.sync_copy(acc.at[pl.ds(s * RB, RB)],
                        out_hbm.at[pl.ds(c * N + s * RB, RB)])


_agg_kernel_args = dict(
    out_type=jax.ShapeDtypeStruct((NC * N, H), jnp.float32),
    mesh=_sc_mesh,
    scratch_types=[
        pltpu.VMEM((EPT, K), jnp.int32),
        pltpu.VMEM((EPT, K), jnp.int32),
        pltpu.VMEM((K, H), jnp.float32),
        pltpu.VMEM_SHARED((ACC_ROWS, H), jnp.float32),
        pltpu.SemaphoreType.DMA,
    ],
)


def _sc_agg_body(hp_hbm, src_hbm, dst_hbm, out_hbm, srcv, dstv, buf, acc, sem):
    c = lax.axis_index("c")
    s = lax.axis_index("s")
    pltpu.sync_copy(src_hbm.at[c, s], srcv)
    pltpu.sync_copy(dst_hbm.at[s], dstv)

    def zb(i, carry):
        buf[i // 8, pl.ds((i % 8) * 16, 16)] = jnp.zeros((16,), jnp.float32)
        return carry
    lax.fori_loop(0, K * H // 16, zb, None)
    for k in range(5):
        pltpu.sync_copy(buf, acc.at[pl.ds(s * 640 + k * K, K)])
    plsc.subcore_barrier()

    def body(j, carry):
        pltpu.async_copy(hp_hbm.at[srcv.at[j]], buf, sem).wait()
        pltpu.sync_copy(buf, acc.at[dstv.at[j]], add=True)
        return carry
    lax.fori_loop(0, EPT, body, None)
    plsc.subcore_barrier()

    @pl.when(s < N // RB)
    def _():
        pltpu.sync_copy(acc.at[pl.ds(s * RB, RB)],
                        out_hbm.at[pl.ds(c * N + s * RB, RB)])


_sc_deg = pl.kernel(_sc_deg_body, **_deg_kernel_args)
_sc_agg = pl.kernel(_sc_agg_body, **_agg_kernel_args)


def _dinv_of(deg_ref):
    d = deg_ref[:, 0:1]
    return jnp.where(d > 0.0, lax.rsqrt(d), 0.0)


def _tc_first_body(x_ref, w_ref, deg_ref, hp_ref):
    dinv = _dinv_of(deg_ref)
    h = jnp.dot(x_ref[...], w_ref[...],
                preferred_element_type=jnp.float32) * dinv
    hp_ref[0] = h[:, :H]
    hp_ref[1] = h[:, H:]


def _tc_mid_body(a_ref, deg_ref, b_ref, w_ref, h_ref, hp_ref):
    dinv = _dinv_of(deg_ref)
    agg = jnp.concatenate([a_ref[0], a_ref[1]], axis=1) * dinv
    hl = jnp.maximum(agg + b_ref[...], 0.0)
    h_ref[...] = hl
    hp = jnp.dot(hl, w_ref[...], preferred_element_type=jnp.float32) * dinv
    hp_ref[0] = hp[:, :H]
    hp_ref[1] = hp[:, H:]


def _tc_last_body(a_ref, deg_ref, b_ref, h_ref):
    dinv = _dinv_of(deg_ref)
    agg = jnp.concatenate([a_ref[0], a_ref[1]], axis=1) * dinv
    h_ref[...] = jnp.maximum(agg + b_ref[...], 0.0)


_spec_rows = pl.BlockSpec((RB, D), lambda i: (i, 0))
_spec_w = pl.BlockSpec((D, D), lambda i: (0, 0))
_spec_deg = pl.BlockSpec((RB, 16), lambda i: (i, 0))
_spec_b = pl.BlockSpec((1, D), lambda i: (0, 0))
_spec_hp = pl.BlockSpec((2, RB, H), lambda i: (0, i, 0))

_tc_first = pl.pallas_call(
    _tc_first_body,
    grid=(N // RB,),
    in_specs=[_spec_rows, _spec_w, _spec_deg],
    out_specs=_spec_hp,
    out_shape=jax.ShapeDtypeStruct((2, N, H), jnp.float32),
)

_tc_mid = pl.pallas_call(
    _tc_mid_body,
    grid=(N // RB,),
    in_specs=[_spec_hp, _spec_deg, _spec_b, _spec_w],
    out_specs=(_spec_rows, _spec_hp),
    out_shape=(jax.ShapeDtypeStruct((N, D), jnp.float32),
               jax.ShapeDtypeStruct((2, N, H), jnp.float32)),
)

_tc_last = pl.pallas_call(
    _tc_last_body,
    grid=(N // RB,),
    in_specs=[_spec_hp, _spec_deg, _spec_b],
    out_specs=_spec_rows,
    out_shape=jax.ShapeDtypeStruct((N, D), jnp.float32),
)


def kernel(x, adj_t, W1, b1, W2, b2, W3, b3):
    src = adj_t[0]
    dst = adj_t[1]
    e = src.shape[0]
    pad = E_PAD - e
    src_p = jnp.concatenate([src, jnp.zeros((pad,), jnp.int32)])
    dst_p = jnp.concatenate([dst, jnp.full((pad,), N, jnp.int32)])
    src2 = jnp.stack([src_p, src_p + N]).reshape(NC, NS, EPT, K)
    dst3 = dst_p.reshape(NS, EPT, K)

    deg16 = _sc_deg(dst3)[:N, :16]
    b1r, b2r, b3r = (b.reshape(1, D) for b in (b1, b2, b3))

    hp1 = _tc_first(x, W1, deg16)
    a1 = _sc_agg(hp1.reshape(NC * N, H), src2, dst3).reshape(NC, N, H)
    h1, hp2 = _tc_mid(a1, deg16, b1r, W2)
    a2 = _sc_agg(hp2.reshape(NC * N, H), src2, dst3).reshape(NC, N, H)
    h2, hp3 = _tc_mid(a2, deg16, b2r, W3)
    a3 = _sc_agg(hp3.reshape(NC * N, H), src2, dst3).reshape(NC, N, H)
    h3 = _tc_last(a3, deg16, b3r)
    return (h1, h2, h3)
